# baseline (speedup 1.0000x reference)
import functools

import jax
import jax.numpy as jnp
import ml_dtypes
import numpy as np
from jax import lax
from jax.experimental import pallas as pl
from jax.experimental.pallas import tpu as pltpu
from jax.experimental.pallas import tpu_sc as plsc

D_MODEL = 1024
MAX_LEN = 2048
BATCH = 4

NUM_CORES = 2
NUM_SUBCORES = 16
NUM_WORKERS = NUM_CORES * NUM_SUBCORES

POS_PER_WORKER = MAX_LEN // NUM_WORKERS
CHUNK = 32
BLOCKS_PER_WORKER = POS_PER_WORKER // CHUNK
STEPS = BLOCKS_PER_WORKER * BATCH
NBUF = 3
LANES = 16
PAIRS = D_MODEL // (2 * LANES)


def _pos_encoding(max_len, d_model):
    pos = np.arange(max_len)[:, np.newaxis]
    depth = np.arange(d_model / 2)[np.newaxis, :] / (d_model / 2)
    angle_rates = 1.0 / 10000 ** depth
    inner = pos * angle_rates
    pe = np.stack((np.sin(inner), np.cos(inner)), axis=2).reshape((max_len, -1))
    pe = np.asarray(pe, dtype=np.float32)
    bits = np.asarray(pe, dtype=ml_dtypes.bfloat16).view(np.uint16)
    grp = bits.reshape(max_len, d_model // 32, 2, 16)
    words = grp[:, :, 0, :].astype(np.uint32) | (
        grp[:, :, 1, :].astype(np.uint32) << 16)
    return words.astype(np.uint32).view(np.int32).reshape(-1)


_POS_ENC_NP = _pos_encoding(MAX_LEN, D_MODEL)
_POS_ENC_DEV = None


@functools.partial(
    pl.kernel,
    mesh=plsc.VectorSubcoreMesh(core_axis_name="c", subcore_axis_name="s"),
    out_type=jax.ShapeDtypeStruct((BATCH, MAX_LEN, D_MODEL), jnp.float32),
    scratch_types=[
        pltpu.VMEM((BATCH, POS_PER_WORKER), jnp.int32),
        pltpu.VMEM((CHUNK * D_MODEL // 2,), jnp.int32),
        pltpu.VMEM((CHUNK, D_MODEL), jnp.float32),
        pltpu.VMEM((CHUNK, D_MODEL), jnp.float32),
        pltpu.VMEM((CHUNK, D_MODEL), jnp.float32),
        pltpu.SemaphoreType.DMA,
        pltpu.SemaphoreType.DMA,
        pltpu.SemaphoreType.DMA,
        pltpu.SemaphoreType.DMA,
        pltpu.SemaphoreType.DMA,
        pltpu.SemaphoreType.DMA,
        pltpu.SemaphoreType.DMA,
        pltpu.SemaphoreType.DMA,
    ],
)
def _sc_embed(idx_hbm, pe_hbm, table_hbm, out_hbm,
              idx_v, pe_v, rows0, rows1, rows2,
              sem_i, sem_pe, sem_g0, sem_g1, sem_g2, sem_o0, sem_o1, sem_o2):
    wid = lax.axis_index("s") * NUM_CORES + lax.axis_index("c")
    pos0 = wid * POS_PER_WORKER
    rows_bufs = (rows0, rows1, rows2)
    g_sems = (sem_g0, sem_g1, sem_g2)
    o_sems = (sem_o0, sem_o1, sem_o2)

    def idx_slice(s):
        return idx_v.at[s % BATCH, pl.ds((s // BATCH) * CHUNK, CHUNK)]

    def out_ref(s):
        return out_hbm.at[s % BATCH, pl.ds(pos0 + (s // BATCH) * CHUNK, CHUNK)]

    idx_h = [
        pltpu.async_copy(idx_hbm.at[b, pl.ds(pos0, POS_PER_WORKER)],
                         idx_v.at[b], sem_i)
        for b in range(BATCH)
    ]
    idx_h[0].wait()
    gather_h = [None] * STEPS
    gather_h[0] = pltpu.async_copy(table_hbm.at[idx_slice(0)], rows0, sem_g0)
    for h in idx_h[1:]:
        h.wait()
    gather_h[1] = pltpu.async_copy(table_hbm.at[idx_slice(1)], rows1, sem_g1)
    pe_h = pltpu.async_copy(
        pe_hbm.at[pl.ds(pos0 * (D_MODEL // 2), CHUNK * D_MODEL // 2)],
        pe_v, sem_pe)

    out_h = [None] * STEPS
    for s in range(STEPS):
        buf = s % NBUF
        if s + 2 < STEPS:
            if s >= 1:
                out_h[s - 1].wait()
            gather_h[s + 2] = pltpu.async_copy(
                table_hbm.at[idx_slice(s + 2)],
                rows_bufs[(s + 2) % NBUF], g_sems[(s + 2) % NBUF])
        if s == 0 or s == BATCH:
            pe_h.wait()
        gather_h[s].wait()

        rv = rows_bufs[buf]

        @plsc.parallel_loop(0, CHUNK, 1, unroll=1)
        def _(j):
            def add_group(kk, _):
                for k8 in range(8):
                    k = kk * 8 + k8
                    w = pe_v[pl.ds(j * (D_MODEL // 2) + k * LANES, LANES)]
                    a = lax.bitcast_convert_type(w << 16, jnp.float32)
                    b = lax.bitcast_convert_type(
                        w & jnp.int32(-65536), jnp.float32)
                    sa = pl.ds(k * 2 * LANES, LANES)
                    sb = pl.ds(k * 2 * LANES + LANES, LANES)
                    rv[j, sa] = rv[j, sa] + a
                    rv[j, sb] = rv[j, sb] + b
                return ()

            lax.fori_loop(0, PAIRS // 8, add_group, ())

        out_h[s] = pltpu.async_copy(rv, out_ref(s), o_sems[buf])

        if s == BATCH - 1:
            pe_h = pltpu.async_copy(
                pe_hbm.at[pl.ds((pos0 + CHUNK) * (D_MODEL // 2),
                                CHUNK * D_MODEL // 2)],
                pe_v, sem_pe)

    for s in (STEPS - 3, STEPS - 2, STEPS - 1):
        out_h[s].wait()


def kernel(inputs, table):
    global _POS_ENC_DEV
    if _POS_ENC_DEV is None:
        _POS_ENC_DEV = jnp.asarray(_POS_ENC_NP)
    return _sc_embed(inputs, _POS_ENC_DEV, table)


# speedup vs baseline: 1.1668x; 1.1668x over previous
import functools

import jax
import jax.numpy as jnp
import ml_dtypes
import numpy as np
from jax import lax
from jax.experimental import pallas as pl
from jax.experimental.pallas import tpu as pltpu
from jax.experimental.pallas import tpu_sc as plsc

D_MODEL = 1024
MAX_LEN = 2048
BATCH = 4

NUM_CORES = 2
NUM_SUBCORES = 16
NUM_WORKERS = NUM_CORES * NUM_SUBCORES

POS_PER_WORKER = MAX_LEN // NUM_WORKERS
CHUNK = 8
BLOCKS_PER_WORKER = POS_PER_WORKER // CHUNK
STEPS = BLOCKS_PER_WORKER * BATCH
NBUF = 8
WINDOW = 8
LANES = 16
PAIRS = D_MODEL // (2 * LANES)
PE_WORDS = D_MODEL // 2


def _pos_encoding(max_len, d_model):
    pos = np.arange(max_len)[:, np.newaxis]
    depth = np.arange(d_model / 2)[np.newaxis, :] / (d_model / 2)
    angle_rates = 1.0 / 10000 ** depth
    inner = pos * angle_rates
    pe = np.stack((np.sin(inner), np.cos(inner)), axis=2).reshape((max_len, -1))
    pe = np.asarray(pe, dtype=np.float32)
    bits = np.asarray(pe, dtype=ml_dtypes.bfloat16).view(np.uint16)
    grp = bits.reshape(max_len, d_model // 32, 2, 16)
    words = grp[:, :, 0, :].astype(np.uint32) | (
        grp[:, :, 1, :].astype(np.uint32) << 16)
    return words.astype(np.uint32).view(np.int32).reshape(-1)


_POS_ENC_NP = _pos_encoding(MAX_LEN, D_MODEL)
_POS_ENC_DEV = None


@functools.partial(
    pl.kernel,
    mesh=plsc.VectorSubcoreMesh(core_axis_name="c", subcore_axis_name="s"),
    out_type=jax.ShapeDtypeStruct((BATCH, MAX_LEN, D_MODEL), jnp.float32),
    scratch_types=(
        [pltpu.VMEM((BATCH, POS_PER_WORKER), jnp.int32)]
        + [pltpu.VMEM((CHUNK * PE_WORDS,), jnp.int32) for _ in range(2)]
        + [pltpu.VMEM((CHUNK, D_MODEL), jnp.float32) for _ in range(NBUF)]
        + [pltpu.SemaphoreType.DMA for _ in range(3 + 2 * NBUF)]
    ),
)
def _sc_embed(idx_hbm, pe_hbm, table_hbm, out_hbm, idx_v, *refs):
    pe_bufs = refs[0:2]
    rows_bufs = refs[2:2 + NBUF]
    sem_i = refs[2 + NBUF]
    pe_sems = refs[3 + NBUF:5 + NBUF]
    g_sems = refs[5 + NBUF:5 + 2 * NBUF]
    o_sems = refs[5 + 2 * NBUF:5 + 3 * NBUF]

    wid = lax.axis_index("s") * NUM_CORES + lax.axis_index("c")
    pos0 = wid * POS_PER_WORKER

    def idx_slice(b, pb):
        return idx_v.at[b, pl.ds(pb * CHUNK, CHUNK)]

    def out_ref(b, pb):
        return out_hbm.at[b, pl.ds(pos0 + pb * CHUNK, CHUNK)]

    def pe_src(pb):
        return pe_hbm.at[pl.ds((pos0 + pb * CHUNK) * PE_WORDS,
                               CHUNK * PE_WORDS)]

    def gather_desc(b, pb, buf):
        return pltpu.make_async_copy(table_hbm.at[idx_slice(b, pb)],
                                     rows_bufs[buf], g_sems[buf])

    def out_desc(b, pb, buf):
        return pltpu.make_async_copy(rows_bufs[buf], out_ref(b, pb),
                                     o_sems[buf])

    PREF = NBUF - 2

    idx_h = [
        pltpu.async_copy(idx_hbm.at[b, pl.ds(pos0, POS_PER_WORKER)],
                         idx_v.at[b], sem_i)
        for b in range(BATCH)
    ]
    for h in idx_h:
        h.wait()
    for v in range(PREF):
        gather_desc(v % BATCH, v // BATCH, v % NBUF).start()
    pltpu.make_async_copy(pe_src(0), pe_bufs[0], pe_sems[0]).start()
    pltpu.make_async_copy(pe_src(1), pe_bufs[1], pe_sems[1]).start()

    def window(h, _):
        for v in range(WINDOW):
            s = h * WINDOW + v
            pb = h * (WINDOW // BATCH) + v // BATCH
            buf = v % NBUF
            pe_par = (v // BATCH) % 2

            @pl.when(s >= 2)
            def _():
                out_desc((v - 2) % BATCH, h * 2 + (v - 2) // BATCH,
                         (v - 2) % NBUF).wait()

            @pl.when(s + PREF < STEPS)
            def _():
                gather_desc((v + PREF) % BATCH, h * 2 + (v + PREF) // BATCH,
                            (v + PREF) % NBUF).start()

            if v % BATCH == 0:
                pltpu.make_async_copy(
                    pe_src(0), pe_bufs[pe_par], pe_sems[pe_par]).wait()
            gather_desc(v % BATCH, pb, buf).wait()
            rv = rows_bufs[buf]
            pe_v = pe_bufs[pe_par]

            @plsc.parallel_loop(0, CHUNK, 1, unroll=1)
            def _(j):
                for k in range(PAIRS):
                    w = pe_v[pl.ds(j * PE_WORDS + k * LANES, LANES)]
                    a = lax.bitcast_convert_type(w << 16, jnp.float32)
                    b = lax.bitcast_convert_type(
                        w & jnp.int32(-65536), jnp.float32)
                    sa = pl.ds(k * 2 * LANES, LANES)
                    sb = pl.ds(k * 2 * LANES + LANES, LANES)
                    rv[j, sa] = rv[j, sa] + a
                    rv[j, sb] = rv[j, sb] + b

            out_desc(v % BATCH, pb, buf).start()

            if v % BATCH == BATCH - 1:
                @pl.when(pb + 2 < BLOCKS_PER_WORKER)
                def _():
                    pltpu.make_async_copy(
                        pe_src(pb + 2), pe_bufs[pe_par],
                        pe_sems[pe_par]).start()
        return ()

    lax.fori_loop(0, STEPS // WINDOW, window, ())
    for s in (STEPS - 2, STEPS - 1):
        out_desc(s % BATCH, s // BATCH, s % NBUF).wait()


def kernel(inputs, table):
    global _POS_ENC_DEV
    if _POS_ENC_DEV is None:
        _POS_ENC_DEV = jnp.asarray(_POS_ENC_NP)
    return _sc_embed(inputs, _POS_ENC_DEV, table)


# speedup vs baseline: 1.4576x; 1.2492x over previous
import functools

import jax
import jax.numpy as jnp
import ml_dtypes
import numpy as np
from jax import lax
from jax.experimental import pallas as pl
from jax.experimental.pallas import tpu as pltpu
from jax.experimental.pallas import tpu_sc as plsc

D_MODEL = 1024
MAX_LEN = 2048
BATCH = 4

NUM_CORES = 2
NUM_SUBCORES = 16
NUM_WORKERS = NUM_CORES * NUM_SUBCORES

POS_PER_WORKER = MAX_LEN // NUM_WORKERS
CHUNK = 32
BLOCKS_PER_WORKER = POS_PER_WORKER // CHUNK
STEPS = BLOCKS_PER_WORKER * BATCH
NBUF = 3
LANES = 16
PAIRS = D_MODEL // (2 * LANES)


def _pos_encoding(max_len, d_model):
    pos = np.arange(max_len)[:, np.newaxis]
    depth = np.arange(d_model / 2)[np.newaxis, :] / (d_model / 2)
    angle_rates = 1.0 / 10000 ** depth
    inner = pos * angle_rates
    pe = np.stack((np.sin(inner), np.cos(inner)), axis=2).reshape((max_len, -1))
    pe = np.asarray(pe, dtype=np.float32)
    bits = np.asarray(pe, dtype=ml_dtypes.bfloat16).view(np.uint16)
    grp = bits.reshape(max_len, d_model // 32, 2, 16)
    words = grp[:, :, 0, :].astype(np.uint32) | (
        grp[:, :, 1, :].astype(np.uint32) << 16)
    return words.astype(np.uint32).view(np.int32).reshape(-1)


_POS_ENC_NP = _pos_encoding(MAX_LEN, D_MODEL)
_POS_ENC_DEV = None


@functools.partial(
    pl.kernel,
    mesh=plsc.VectorSubcoreMesh(core_axis_name="c", subcore_axis_name="s"),
    out_type=jax.ShapeDtypeStruct((BATCH, MAX_LEN, D_MODEL), jnp.float32),
    scratch_types=[
        pltpu.VMEM((BATCH, POS_PER_WORKER), jnp.int32),
        pltpu.VMEM((CHUNK * D_MODEL // 2,), jnp.int32),
        pltpu.VMEM((CHUNK, D_MODEL), jnp.float32),
        pltpu.VMEM((CHUNK, D_MODEL), jnp.float32),
        pltpu.VMEM((CHUNK, D_MODEL), jnp.float32),
        pltpu.SemaphoreType.DMA,
        pltpu.SemaphoreType.DMA,
        pltpu.SemaphoreType.DMA,
        pltpu.SemaphoreType.DMA,
        pltpu.SemaphoreType.DMA,
        pltpu.SemaphoreType.DMA,
        pltpu.SemaphoreType.DMA,
        pltpu.SemaphoreType.DMA,
    ],
)
def _sc_embed(idx_hbm, pe_hbm, table_hbm, out_hbm,
              idx_v, pe_v, rows0, rows1, rows2,
              sem_i, sem_pe, sem_g0, sem_g1, sem_g2, sem_o0, sem_o1, sem_o2):
    wid = lax.axis_index("s") * NUM_CORES + lax.axis_index("c")
    pos0 = wid * POS_PER_WORKER
    rows_bufs = (rows0, rows1, rows2)
    g_sems = (sem_g0, sem_g1, sem_g2)
    o_sems = (sem_o0, sem_o1, sem_o2)

    def idx_slice(s):
        return idx_v.at[s % BATCH, pl.ds((s // BATCH) * CHUNK, CHUNK)]

    def out_ref(s):
        return out_hbm.at[s % BATCH, pl.ds(pos0 + (s // BATCH) * CHUNK, CHUNK)]

    idx_h = [
        pltpu.async_copy(idx_hbm.at[b, pl.ds(pos0, POS_PER_WORKER)],
                         idx_v.at[b], sem_i)
        for b in range(BATCH)
    ]
    idx_h[0].wait()
    gather_h = [None] * STEPS
    gather_h[0] = pltpu.async_copy(table_hbm.at[idx_slice(0)], rows0, sem_g0)
    for h in idx_h[1:]:
        h.wait()
    gather_h[1] = pltpu.async_copy(table_hbm.at[idx_slice(1)], rows1, sem_g1)
    pe_h = pltpu.async_copy(
        pe_hbm.at[pl.ds(pos0 * (D_MODEL // 2), CHUNK * D_MODEL // 2)],
        pe_v, sem_pe)

    out_h = [None] * STEPS
    for s in range(STEPS):
        buf = s % NBUF
        if s == 0 or s == BATCH:
            pe_h.wait()
        gather_h[s].wait()

        rv = rows_bufs[buf]

        @plsc.parallel_loop(0, CHUNK, 1, unroll=1)
        def _(j):
            for k in range(PAIRS):
                w = pe_v[pl.ds(j * (D_MODEL // 2) + k * LANES, LANES)]
                a = lax.bitcast_convert_type(w << 16, jnp.float32)
                b = lax.bitcast_convert_type(w & jnp.int32(-65536), jnp.float32)
                sa = pl.ds(k * 2 * LANES, LANES)
                sb = pl.ds(k * 2 * LANES + LANES, LANES)
                rv[j, sa] = rv[j, sa] + a
                rv[j, sb] = rv[j, sb] + b

        if s + 2 < STEPS:
            if s >= 1:
                out_h[s - 1].wait()
            gather_h[s + 2] = pltpu.async_copy(
                table_hbm.at[idx_slice(s + 2)],
                rows_bufs[(s + 2) % NBUF], g_sems[(s + 2) % NBUF])

        out_h[s] = pltpu.async_copy(rv, out_ref(s), o_sems[buf])

        if s == BATCH - 1:
            pe_h = pltpu.async_copy(
                pe_hbm.at[pl.ds((pos0 + CHUNK) * (D_MODEL // 2),
                                CHUNK * D_MODEL // 2)],
                pe_v, sem_pe)

    for s in (STEPS - 3, STEPS - 2, STEPS - 1):
        out_h[s].wait()


def kernel(inputs, table):
    global _POS_ENC_DEV
    if _POS_ENC_DEV is None:
        _POS_ENC_DEV = jnp.asarray(_POS_ENC_NP)
    return _sc_embed(inputs, _POS_ENC_DEV, table)
